# baseline (speedup 1.0000x reference)
import functools

import jax
import jax.numpy as jnp
from jax import lax
from jax.experimental import pallas as pl
from jax.experimental.pallas import tpu as pltpu
from jax.experimental.pallas import tpu_sc as plsc

VOCAB = 100000
DIM = 64
BATCH = 16384

NUM_CORES = 2
NUM_SUBCORES = 16
LANES = 16
NW = NUM_CORES * NUM_SUBCORES
BPW = BATCH // NW
CHUNK = 128
NCHUNK = BPW // CHUNK


def _sqrt_vec(x):
    xi = lax.bitcast_convert_type(x, jnp.int32)
    y = lax.bitcast_convert_type(jnp.int32(0x5F3759DF) - (xi >> 1), jnp.float32)
    for _ in range(3):
        y = y * (1.5 - 0.5 * x * y * y)
    return x * y


def _sc_kernel(w_idx_hbm, c_idx_hbm, w_hbm, c_hbm,
               score_hbm, sw_hbm, sc_hbm,
               widx_v, cidx_v, wrows_v, crows_v,
               s_v, sw_v, sc_v, sem):
    wid = lax.axis_index("s") * NUM_CORES + lax.axis_index("c")
    base = wid * BPW

    for k in range(NCHUNK):
        pltpu.sync_copy(w_idx_hbm.at[pl.ds(base + k * CHUNK, CHUNK)],
                        widx_v.at[k])
        pltpu.sync_copy(c_idx_hbm.at[pl.ds(base + k * CHUNK, CHUNK)],
                        cidx_v.at[k])

    copies = []
    for k in range(NCHUNK):
        copies.append(pltpu.async_copy(
            w_hbm.at[widx_v.at[k]], wrows_v.at[pl.ds(k * CHUNK, CHUNK)], sem))
        copies.append(pltpu.async_copy(
            c_hbm.at[cidx_v.at[k]], crows_v.at[pl.ds(k * CHUNK, CHUNK)], sem))
    for cp in copies:
        cp.wait()

    lane_iota = lax.iota(jnp.int32, LANES)

    def group_body(g, _):
        rs = jnp.zeros((LANES,), jnp.float32)
        rw = jnp.zeros((LANES,), jnp.float32)
        rc = jnp.zeros((LANES,), jnp.float32)
        for r in range(LANES):
            row = g * LANES + r
            acc_s = jnp.zeros((LANES,), jnp.float32)
            acc_w = jnp.zeros((LANES,), jnp.float32)
            acc_c = jnp.zeros((LANES,), jnp.float32)
            for kk in range(DIM // LANES):
                wv = wrows_v[row, pl.ds(kk * LANES, LANES)]
                cv = crows_v[row, pl.ds(kk * LANES, LANES)]
                acc_s = acc_s + wv * cv
                acc_w = acc_w + wv * wv
                acc_c = acc_c + cv * cv
            m = lane_iota == r
            rs = jnp.where(m, jnp.sum(acc_s), rs)
            rw = jnp.where(m, jnp.sum(acc_w), rw)
            rc = jnp.where(m, jnp.sum(acc_c), rc)
        sl = pl.ds(g * LANES, LANES)
        s_v[sl] = rs
        sw_v[sl] = _sqrt_vec(rw)
        sc_v[sl] = _sqrt_vec(rc)
        return 0

    lax.fori_loop(0, BPW // LANES, group_body, 0)

    pltpu.sync_copy(s_v, score_hbm.at[pl.ds(base, BPW)])
    pltpu.sync_copy(sw_v, sw_hbm.at[pl.ds(base, BPW)])
    pltpu.sync_copy(sc_v, sc_hbm.at[pl.ds(base, BPW)])


_mesh = plsc.VectorSubcoreMesh(
    core_axis_name="c", subcore_axis_name="s",
    num_cores=NUM_CORES, num_subcores=NUM_SUBCORES)

_sc_call = functools.partial(
    pl.kernel,
    out_type=(
        jax.ShapeDtypeStruct((BATCH,), jnp.float32),
        jax.ShapeDtypeStruct((BATCH,), jnp.float32),
        jax.ShapeDtypeStruct((BATCH,), jnp.float32),
    ),
    mesh=_mesh,
    compiler_params=pltpu.CompilerParams(
        needs_layout_passes=False, use_tc_tiling_on_sc=False),
    scratch_types=[
        pltpu.VMEM((NCHUNK, CHUNK), jnp.int32),
        pltpu.VMEM((NCHUNK, CHUNK), jnp.int32),
        pltpu.VMEM((BPW, DIM), jnp.float32),
        pltpu.VMEM((BPW, DIM), jnp.float32),
        pltpu.VMEM((BPW,), jnp.float32),
        pltpu.VMEM((BPW,), jnp.float32),
        pltpu.VMEM((BPW,), jnp.float32),
        pltpu.SemaphoreType.DMA,
    ],
)(_sc_kernel)


@jax.jit
def kernel(w_idx, c_idx, W, C):
    w_idx = w_idx.astype(jnp.int32)
    c_idx = c_idx.astype(jnp.int32)
    return _sc_call(w_idx, c_idx, W, C)


# speedup vs baseline: 1.0040x; 1.0040x over previous
import functools

import jax
import jax.numpy as jnp
from jax import lax
from jax.experimental import pallas as pl
from jax.experimental.pallas import tpu as pltpu
from jax.experimental.pallas import tpu_sc as plsc

VOCAB = 100000
DIM = 64
BATCH = 16384

NUM_CORES = 2
NUM_SUBCORES = 16
LANES = 16
NW = NUM_CORES * NUM_SUBCORES
BPW = BATCH // NW
CHUNK = 128
NCHUNK = BPW // CHUNK

_COMPILER_PARAMS = pltpu.CompilerParams(
    needs_layout_passes=False, use_tc_tiling_on_sc=False)

_mesh = plsc.VectorSubcoreMesh(
    core_axis_name="c", subcore_axis_name="s",
    num_cores=NUM_CORES, num_subcores=NUM_SUBCORES)


def _worker_base():
    wid = lax.axis_index("s") * NUM_CORES + lax.axis_index("c")
    return wid * BPW


def _sqrt_vec(x):
    xi = lax.bitcast_convert_type(x, jnp.int32)
    y = lax.bitcast_convert_type(jnp.int32(0x5F3759DF) - (xi >> 1), jnp.float32)
    for _ in range(3):
        y = y * (1.5 - 0.5 * x * y * y)
    return x * y


def _stage_and_gather(idx_hbm, table_hbm, idx_v, rows_v, sem, base):
    for k in range(NCHUNK):
        pltpu.sync_copy(idx_hbm.at[pl.ds(base + k * CHUNK, CHUNK)],
                        idx_v.at[k])
    copies = []
    for k in range(NCHUNK):
        copies.append(pltpu.async_copy(
            table_hbm.at[idx_v.at[k]], rows_v.at[pl.ds(k * CHUNK, CHUNK)],
            sem))
    return copies


def _w_kernel(w_idx_hbm, w_hbm, sw_hbm, wg_hbm,
              widx_v, wrows_v, sw_v, sem):
    base = _worker_base()
    for cp in _stage_and_gather(w_idx_hbm, w_hbm, widx_v, wrows_v, sem, base):
        cp.wait()

    lane_iota = lax.iota(jnp.int32, LANES)

    def group_body(g, _):
        rw = jnp.zeros((LANES,), jnp.float32)
        for r in range(LANES):
            row = g * LANES + r
            acc_w = jnp.zeros((LANES,), jnp.float32)
            for kk in range(DIM // LANES):
                wv = wrows_v[row, pl.ds(kk * LANES, LANES)]
                acc_w = acc_w + wv * wv
            rw = jnp.where(lane_iota == r, jnp.sum(acc_w), rw)
        sw_v[pl.ds(g * LANES, LANES)] = _sqrt_vec(rw)
        return 0

    lax.fori_loop(0, BPW // LANES, group_body, 0)

    pltpu.sync_copy(sw_v, sw_hbm.at[pl.ds(base, BPW)])
    pltpu.sync_copy(wrows_v, wg_hbm.at[pl.ds(base, BPW)])


def _c_kernel(c_idx_hbm, c_hbm, wg_hbm, score_hbm, sc_hbm,
              cidx_v, crows_v, wrows_v, s_v, sc_v, sem, wsem):
    base = _worker_base()
    copies = _stage_and_gather(c_idx_hbm, c_hbm, cidx_v, crows_v, sem, base)
    wcp = pltpu.async_copy(wg_hbm.at[pl.ds(base, BPW)], wrows_v, wsem)
    for cp in copies:
        cp.wait()
    wcp.wait()

    lane_iota = lax.iota(jnp.int32, LANES)

    def group_body(g, _):
        rs = jnp.zeros((LANES,), jnp.float32)
        rc = jnp.zeros((LANES,), jnp.float32)
        for r in range(LANES):
            row = g * LANES + r
            acc_s = jnp.zeros((LANES,), jnp.float32)
            acc_c = jnp.zeros((LANES,), jnp.float32)
            for kk in range(DIM // LANES):
                wv = wrows_v[row, pl.ds(kk * LANES, LANES)]
                cv = crows_v[row, pl.ds(kk * LANES, LANES)]
                acc_s = acc_s + wv * cv
                acc_c = acc_c + cv * cv
            m = lane_iota == r
            rs = jnp.where(m, jnp.sum(acc_s), rs)
            rc = jnp.where(m, jnp.sum(acc_c), rc)
        sl = pl.ds(g * LANES, LANES)
        s_v[sl] = rs
        sc_v[sl] = _sqrt_vec(rc)
        return 0

    lax.fori_loop(0, BPW // LANES, group_body, 0)

    pltpu.sync_copy(s_v, score_hbm.at[pl.ds(base, BPW)])
    pltpu.sync_copy(sc_v, sc_hbm.at[pl.ds(base, BPW)])


_w_call = functools.partial(
    pl.kernel,
    out_type=(
        jax.ShapeDtypeStruct((BATCH,), jnp.float32),
        jax.ShapeDtypeStruct((BATCH, DIM), jnp.float32),
    ),
    mesh=_mesh,
    compiler_params=_COMPILER_PARAMS,
    scratch_types=[
        pltpu.VMEM((NCHUNK, CHUNK), jnp.int32),
        pltpu.VMEM((BPW, DIM), jnp.float32),
        pltpu.VMEM((BPW,), jnp.float32),
        pltpu.SemaphoreType.DMA,
    ],
)(_w_kernel)

_c_call = functools.partial(
    pl.kernel,
    out_type=(
        jax.ShapeDtypeStruct((BATCH,), jnp.float32),
        jax.ShapeDtypeStruct((BATCH,), jnp.float32),
    ),
    mesh=_mesh,
    compiler_params=_COMPILER_PARAMS,
    scratch_types=[
        pltpu.VMEM((NCHUNK, CHUNK), jnp.int32),
        pltpu.VMEM((BPW, DIM), jnp.float32),
        pltpu.VMEM((BPW, DIM), jnp.float32),
        pltpu.VMEM((BPW,), jnp.float32),
        pltpu.VMEM((BPW,), jnp.float32),
        pltpu.SemaphoreType.DMA,
        pltpu.SemaphoreType.DMA,
    ],
)(_c_kernel)


@jax.jit
def kernel(w_idx, c_idx, W, C):
    w_idx = w_idx.astype(jnp.int32)
    c_idx = c_idx.astype(jnp.int32)
    score_w, wg = _w_call(w_idx, W)
    score, score_c = _c_call(c_idx, C, wg)
    return (score, score_w, score_c)
